# baseline (speedup 1.0000x reference)
import functools

import jax
import jax.numpy as jnp
from jax import lax
from jax.experimental import pallas as pl
from jax.experimental.pallas import tpu as pltpu
from jax.experimental.pallas import tpu_sc as plsc

N = 10000
D = 128
KNN = 16
NP = 10240
EPS = 1e-5
BLK = 1000
BQ = 128
PAD_H = 1e18
NEG = -1e36

NW = 32
NODES_W = NP // NW
SB = 8
EB = SB * KNN


def _f32(x):
    return x.astype(jnp.float32)


def _k1(x_ref, w1t_ref, h_ref, st_ref):
    i = pl.program_id(0)
    h = jnp.dot(x_ref[...], w1t_ref[...], preferred_element_type=jnp.float32)
    h_ref[...] = h

    @pl.when(i == 0)
    def _():
        st_ref[...] = jnp.zeros_like(st_ref)

    st_ref[0:1, :] += jnp.sum(h, axis=0, keepdims=True)
    st_ref[1:2, :] += jnp.sum(h * h, axis=0, keepdims=True)


def _k2(h_ref, p_ref, w2ft_ref, w2xt_ref, st_ref, g1_ref, b1_ref,
        u_ref, v_ref):
    m1 = st_ref[0:1, :] / N
    var1 = st_ref[1:2, :] / N - m1 * m1
    a1 = lax.rsqrt(var1 + EPS) * g1_ref[...]
    hn = jnp.maximum((h_ref[...] - m1) * a1 + b1_ref[...], 0.0)
    v = jnp.dot(p_ref[...], w2xt_ref[...], preferred_element_type=jnp.float32)
    u_ref[...] = jnp.dot(hn, w2ft_ref[...],
                         preferred_element_type=jnp.float32) + v
    v_ref[...] = v


NSEG = 320
GSEG = NP // NSEG


def _k3(qa_ref, pta_ref, idx_ref):
    pta = pta_ref[...]
    pn = 0.25 * jnp.sum(pta * pta, axis=0, keepdims=True)
    score = jnp.dot(qa_ref[...], pta,
                    preferred_element_type=jnp.float32) - pn

    iota = lax.broadcasted_iota(jnp.int32, (BQ, NSEG), 1)
    m1 = score[:, 0:NSEG]
    c1 = iota
    m2 = jnp.full((BQ, NSEG), NEG, jnp.float32)
    m3 = m2
    c2 = jnp.zeros((BQ, NSEG), jnp.int32)
    c3 = c2
    for s in range(1, GSEG):
        sl = score[:, s * NSEG:(s + 1) * NSEG]
        cc = iota + (s * NSEG)
        is1 = sl > m1
        d1 = jnp.where(is1, m1, sl)
        dc1 = jnp.where(is1, c1, cc)
        m1 = jnp.where(is1, sl, m1)
        c1 = jnp.where(is1, cc, c1)
        is2 = d1 > m2
        d2 = jnp.where(is2, m2, d1)
        dc2 = jnp.where(is2, c2, dc1)
        m2 = jnp.where(is2, d1, m2)
        c2 = jnp.where(is2, dc1, c2)
        is3 = d2 > m3
        m3 = jnp.where(is3, d2, m3)
        c3 = jnp.where(is3, dc2, c3)

    cnt = jnp.zeros((BQ, NSEG), jnp.int32)
    cols = []
    for _ in range(KNN):
        m = jnp.max(m1, axis=1, keepdims=True)
        j = jnp.min(jnp.where(m1 == m, iota, NSEG), axis=1, keepdims=True)
        oh = iota == j
        csel = jnp.where(cnt == 0, c1, jnp.where(cnt == 1, c2, c3))
        nxt = jnp.where(cnt == 0, m2, jnp.where(cnt == 1, m3, NEG))
        cols.append(jnp.max(jnp.where(oh, csel, -1), axis=1, keepdims=True))
        m1 = jnp.where(oh, nxt, m1)
        cnt = cnt + oh.astype(jnp.int32)
    idx_ref[...] = jnp.concatenate(cols, axis=1)


def _k4_body(u_hbm, idx_hbm, m_hbm, n_hbm, s_hbm, q_hbm,
             idx_v, rows_v, mv, nv, sv, qv, sem):
    wid = lax.axis_index("s") * 2 + lax.axis_index("c")
    node_base = wid * NODES_W

    def sub_batch(b, carry):
        nb = node_base + b * SB
        pltpu.sync_copy(idx_hbm.at[pl.ds(nb * KNN, EB)], idx_v)
        pltpu.async_copy(u_hbm.at[idx_v], rows_v, sem).wait()

        def node(i, carry2):
            base = i * KNN
            for c in range(D // 16):
                sl = pl.ds(c * 16, 16)
                r0 = rows_v[base, sl]
                am, an, asum, asq = r0, r0, r0, r0 * r0
                for s in range(1, KNN):
                    r = rows_v[base + s, sl]
                    am = jnp.maximum(am, r)
                    an = jnp.minimum(an, r)
                    asum = asum + r
                    asq = asq + r * r
                mv[i, sl] = am
                nv[i, sl] = an
                sv[i, sl] = asum
                qv[i, sl] = asq
            return carry2

        lax.fori_loop(0, SB, node, 0)
        pltpu.sync_copy(mv, m_hbm.at[pl.ds(nb, SB)])
        pltpu.sync_copy(nv, n_hbm.at[pl.ds(nb, SB)])
        pltpu.sync_copy(sv, s_hbm.at[pl.ds(nb, SB)])
        pltpu.sync_copy(qv, q_hbm.at[pl.ds(nb, SB)])
        return carry

    lax.fori_loop(0, NODES_W // SB, sub_batch, 0)


def _gather_reduce(u, idx_flat):
    mesh = plsc.VectorSubcoreMesh(core_axis_name="c", subcore_axis_name="s")
    fn = functools.partial(
        pl.kernel,
        mesh=mesh,
        out_type=[jax.ShapeDtypeStruct((NP, D), jnp.float32)] * 4,
        scratch_types=[
            pltpu.VMEM((EB,), jnp.int32),
            pltpu.VMEM((EB, D), jnp.float32),
            pltpu.VMEM((SB, D), jnp.float32),
            pltpu.VMEM((SB, D), jnp.float32),
            pltpu.VMEM((SB, D), jnp.float32),
            pltpu.VMEM((SB, D), jnp.float32),
            pltpu.SemaphoreType.DMA,
        ],
    )(_k4_body)
    return fn(u, idx_flat)


def _k5(s_ref, q_ref, v_ref, out_ref):
    i = pl.program_id(0)

    @pl.when(i == 0)
    def _():
        out_ref[...] = jnp.zeros_like(out_ref)

    s = s_ref[...]
    v = v_ref[...]
    out_ref[0:1, :] += jnp.sum(s, axis=0, keepdims=True)
    out_ref[1:2, :] += jnp.sum(q_ref[...], axis=0, keepdims=True)
    out_ref[2:3, :] += jnp.sum(v, axis=0, keepdims=True)
    out_ref[3:4, :] += jnp.sum(v * v, axis=0, keepdims=True)
    out_ref[4:5, :] += jnp.sum(v * s, axis=0, keepdims=True)


def _k6(m_ref, n_ref, v_ref, w3t_ref, s2_ref, g2_ref, b2_ref,
        t_ref, st_ref):
    i = pl.program_id(0)
    nk = float(N * KNN)
    s_sum = s2_ref[0:1, :]
    q_sum = s2_ref[1:2, :]
    v_sum = s2_ref[2:3, :]
    v2_sum = s2_ref[3:4, :]
    vs_sum = s2_ref[4:5, :]
    mean2 = (s_sum - KNN * v_sum) / nk
    e2 = (q_sum - 2.0 * vs_sum + KNN * v2_sum) / nk
    var2 = e2 - mean2 * mean2
    a2 = lax.rsqrt(var2 + EPS) * g2_ref[...]
    v = v_ref[...]
    hi = a2 * (m_ref[...] - v - mean2)
    lo = a2 * (n_ref[...] - v - mean2)
    pooled = jnp.maximum(jnp.maximum(hi, lo) + b2_ref[...], 0.0)
    t = jnp.dot(pooled, w3t_ref[...], preferred_element_type=jnp.float32)
    t_ref[...] = t

    @pl.when(i == 0)
    def _():
        st_ref[...] = jnp.zeros_like(st_ref)

    st_ref[0:1, :] += jnp.sum(t, axis=0, keepdims=True)
    st_ref[1:2, :] += jnp.sum(t * t, axis=0, keepdims=True)


def _k7(t_ref, x_ref, st_ref, g3_ref, b3_ref, o_ref):
    m3 = st_ref[0:1, :] / N
    var3 = st_ref[1:2, :] / N - m3 * m3
    a3 = lax.rsqrt(var3 + EPS) * g3_ref[...]
    o_ref[...] = jnp.maximum(
        (t_ref[...] - m3) * a3 + b3_ref[...] + x_ref[...], 0.0)


def kernel(p, x, o, W1, g1, b1, W2, g2, b2, W3, g3, b3):
    p = _f32(p)
    x = _f32(x)
    f32 = jnp.float32

    row = lambda a: a.reshape(1, D)
    rowspec = pl.BlockSpec((1, D), lambda i: (0, 0))
    full = lambda r, c: pl.BlockSpec((r, c), lambda i: (0, 0))
    blk = pl.BlockSpec((BLK, D), lambda i: (i, 0))
    stspec = pl.BlockSpec((8, D), lambda i: (0, 0))

    h, st1 = pl.pallas_call(
        _k1,
        grid=(N // BLK,),
        in_specs=[blk, full(D, D)],
        out_specs=[blk, stspec],
        out_shape=[jax.ShapeDtypeStruct((N, D), f32),
                   jax.ShapeDtypeStruct((8, D), f32)],
    )(x, W1.T)

    p_pad = jnp.zeros((N, D), f32).at[:, :3].set(p)
    w2xt = jnp.zeros((D, D), f32).at[:3, :].set(W2[:, :3].T)
    u, v = pl.pallas_call(
        _k2,
        grid=(N // BLK,),
        in_specs=[blk, blk, full(D, D), full(D, D), stspec,
                  rowspec, rowspec],
        out_specs=[blk, blk],
        out_shape=[jax.ShapeDtypeStruct((N, D), f32),
                   jax.ShapeDtypeStruct((N, D), f32)],
    )(h, p_pad, W2[:, 3:].T, w2xt, st1, row(g1), row(b1))

    qa = jnp.zeros((NP, 8), f32).at[:N, :3].set(p)
    pta = jnp.zeros((8, NP), f32).at[:3, :N].set(2.0 * p.T)
    pta = pta.at[3, N:].set(PAD_H)
    idx = pl.pallas_call(
        _k3,
        grid=(NP // BQ,),
        in_specs=[pl.BlockSpec((BQ, 8), lambda i: (i, 0)), full(8, NP)],
        out_specs=pl.BlockSpec((BQ, KNN), lambda i: (i, 0)),
        out_shape=jax.ShapeDtypeStruct((NP, KNN), jnp.int32),
    )(qa, pta)

    mx, mn, sm, sq = _gather_reduce(u, idx.reshape(-1))

    sums2 = pl.pallas_call(
        _k5,
        grid=(N // BLK,),
        in_specs=[blk, blk, blk],
        out_specs=stspec,
        out_shape=jax.ShapeDtypeStruct((8, D), f32),
    )(sm[:N], sq[:N], v)

    t, st3 = pl.pallas_call(
        _k6,
        grid=(N // BLK,),
        in_specs=[blk, blk, blk, full(D, D), stspec, rowspec, rowspec],
        out_specs=[blk, stspec],
        out_shape=[jax.ShapeDtypeStruct((N, D), f32),
                   jax.ShapeDtypeStruct((8, D), f32)],
    )(mx[:N], mn[:N], v, W3.T, sums2, row(g2), row(b2))

    out = pl.pallas_call(
        _k7,
        grid=(N // BLK,),
        in_specs=[blk, blk, stspec, rowspec, rowspec],
        out_specs=blk,
        out_shape=jax.ShapeDtypeStruct((N, D), f32),
    )(t, x, st3, row(g3), row(b3))

    return (p, out, o)


# speedup vs baseline: 1.2895x; 1.2895x over previous
import functools

import jax
import jax.numpy as jnp
from jax import lax
from jax.experimental import pallas as pl
from jax.experimental.pallas import tpu as pltpu
from jax.experimental.pallas import tpu_sc as plsc

N = 10000
D = 128
KNN = 16
NP = 10240
EPS = 1e-5
BLK = 1000
BQ = 128
PAD_H = 1e18
NEG = -1e36

NW = 32
NODES_W = NP // NW
SB = 8
EB = SB * KNN


def _f32(x):
    return x.astype(jnp.float32)


def _k1(x_ref, w1t_ref, h_ref, st_ref):
    i = pl.program_id(0)
    h = jnp.dot(x_ref[...], w1t_ref[...], preferred_element_type=jnp.float32)
    h_ref[...] = h

    @pl.when(i == 0)
    def _():
        st_ref[...] = jnp.zeros_like(st_ref)

    st_ref[0:1, :] += jnp.sum(h, axis=0, keepdims=True)
    st_ref[1:2, :] += jnp.sum(h * h, axis=0, keepdims=True)


def _k2(h_ref, p_ref, w2ft_ref, w2xt_ref, st_ref, g1_ref, b1_ref,
        u_ref, v_ref):
    m1 = st_ref[0:1, :] / N
    var1 = st_ref[1:2, :] / N - m1 * m1
    a1 = lax.rsqrt(var1 + EPS) * g1_ref[...]
    hn = jnp.maximum((h_ref[...] - m1) * a1 + b1_ref[...], 0.0)
    v = jnp.dot(p_ref[...], w2xt_ref[...], preferred_element_type=jnp.float32)
    u_ref[...] = jnp.dot(hn, w2ft_ref[...],
                         preferred_element_type=jnp.float32) + v
    v_ref[...] = v


NSEG = 1280
GSEG = NP // NSEG


def _k3(qa_ref, pta_ref, idx_ref):
    pta = pta_ref[...]
    pn = 0.25 * jnp.sum(pta * pta, axis=0, keepdims=True)
    score = jnp.dot(qa_ref[...], pta,
                    preferred_element_type=jnp.float32) - pn

    iota = lax.broadcasted_iota(jnp.int32, (BQ, NSEG), 1)
    m1 = score[:, 0:NSEG]
    c1 = iota
    m2 = jnp.full((BQ, NSEG), NEG, jnp.float32)
    m3 = m2
    c2 = jnp.zeros((BQ, NSEG), jnp.int32)
    c3 = c2
    for s in range(1, GSEG):
        sl = score[:, s * NSEG:(s + 1) * NSEG]
        cc = iota + (s * NSEG)
        is1 = sl > m1
        d1 = jnp.where(is1, m1, sl)
        dc1 = jnp.where(is1, c1, cc)
        m1 = jnp.where(is1, sl, m1)
        c1 = jnp.where(is1, cc, c1)
        is2 = d1 > m2
        d2 = jnp.where(is2, m2, d1)
        dc2 = jnp.where(is2, c2, dc1)
        m2 = jnp.where(is2, d1, m2)
        c2 = jnp.where(is2, dc1, c2)
        is3 = d2 > m3
        m3 = jnp.where(is3, d2, m3)
        c3 = jnp.where(is3, dc2, c3)

    cnt = jnp.zeros((BQ, NSEG), jnp.int32)
    cols = []
    for _ in range(KNN):
        m = jnp.max(m1, axis=1, keepdims=True)
        j = jnp.min(jnp.where(m1 == m, iota, NSEG), axis=1, keepdims=True)
        oh = iota == j
        csel = jnp.where(cnt == 0, c1, jnp.where(cnt == 1, c2, c3))
        nxt = jnp.where(cnt == 0, m2, jnp.where(cnt == 1, m3, NEG))
        cols.append(jnp.max(jnp.where(oh, csel, -1), axis=1, keepdims=True))
        m1 = jnp.where(oh, nxt, m1)
        cnt = cnt + oh.astype(jnp.int32)
    idx_ref[...] = jnp.concatenate(cols, axis=1)


def _k4_body(u_hbm, idx_hbm, m_hbm, n_hbm, s_hbm, q_hbm,
             idx_v, rows_v, mv, nv, sv, qv, sem):
    wid = lax.axis_index("s") * 2 + lax.axis_index("c")
    node_base = wid * NODES_W

    def sub_batch(b, carry):
        nb = node_base + b * SB
        pltpu.sync_copy(idx_hbm.at[pl.ds(nb * KNN, EB)], idx_v)
        pltpu.async_copy(u_hbm.at[idx_v], rows_v, sem).wait()

        def node(i, carry2):
            base = i * KNN
            for c in range(D // 16):
                sl = pl.ds(c * 16, 16)
                r0 = rows_v[base, sl]
                am, an, asum, asq = r0, r0, r0, r0 * r0
                for s in range(1, KNN):
                    r = rows_v[base + s, sl]
                    am = jnp.maximum(am, r)
                    an = jnp.minimum(an, r)
                    asum = asum + r
                    asq = asq + r * r
                mv[i, sl] = am
                nv[i, sl] = an
                sv[i, sl] = asum
                qv[i, sl] = asq
            return carry2

        lax.fori_loop(0, SB, node, 0)
        pltpu.sync_copy(mv, m_hbm.at[pl.ds(nb, SB)])
        pltpu.sync_copy(nv, n_hbm.at[pl.ds(nb, SB)])
        pltpu.sync_copy(sv, s_hbm.at[pl.ds(nb, SB)])
        pltpu.sync_copy(qv, q_hbm.at[pl.ds(nb, SB)])
        return carry

    lax.fori_loop(0, NODES_W // SB, sub_batch, 0)


def _gather_reduce(u, idx_flat):
    mesh = plsc.VectorSubcoreMesh(core_axis_name="c", subcore_axis_name="s")
    fn = functools.partial(
        pl.kernel,
        mesh=mesh,
        out_type=[jax.ShapeDtypeStruct((NP, D), jnp.float32)] * 4,
        scratch_types=[
            pltpu.VMEM((EB,), jnp.int32),
            pltpu.VMEM((EB, D), jnp.float32),
            pltpu.VMEM((SB, D), jnp.float32),
            pltpu.VMEM((SB, D), jnp.float32),
            pltpu.VMEM((SB, D), jnp.float32),
            pltpu.VMEM((SB, D), jnp.float32),
            pltpu.SemaphoreType.DMA,
        ],
    )(_k4_body)
    return fn(u, idx_flat)


def _k5(s_ref, q_ref, v_ref, out_ref):
    i = pl.program_id(0)

    @pl.when(i == 0)
    def _():
        out_ref[...] = jnp.zeros_like(out_ref)

    s = s_ref[...]
    v = v_ref[...]
    out_ref[0:1, :] += jnp.sum(s, axis=0, keepdims=True)
    out_ref[1:2, :] += jnp.sum(q_ref[...], axis=0, keepdims=True)
    out_ref[2:3, :] += jnp.sum(v, axis=0, keepdims=True)
    out_ref[3:4, :] += jnp.sum(v * v, axis=0, keepdims=True)
    out_ref[4:5, :] += jnp.sum(v * s, axis=0, keepdims=True)


def _k6(m_ref, n_ref, v_ref, w3t_ref, s2_ref, g2_ref, b2_ref,
        t_ref, st_ref):
    i = pl.program_id(0)
    nk = float(N * KNN)
    s_sum = s2_ref[0:1, :]
    q_sum = s2_ref[1:2, :]
    v_sum = s2_ref[2:3, :]
    v2_sum = s2_ref[3:4, :]
    vs_sum = s2_ref[4:5, :]
    mean2 = (s_sum - KNN * v_sum) / nk
    e2 = (q_sum - 2.0 * vs_sum + KNN * v2_sum) / nk
    var2 = e2 - mean2 * mean2
    a2 = lax.rsqrt(var2 + EPS) * g2_ref[...]
    v = v_ref[...]
    hi = a2 * (m_ref[...] - v - mean2)
    lo = a2 * (n_ref[...] - v - mean2)
    pooled = jnp.maximum(jnp.maximum(hi, lo) + b2_ref[...], 0.0)
    t = jnp.dot(pooled, w3t_ref[...], preferred_element_type=jnp.float32)
    t_ref[...] = t

    @pl.when(i == 0)
    def _():
        st_ref[...] = jnp.zeros_like(st_ref)

    st_ref[0:1, :] += jnp.sum(t, axis=0, keepdims=True)
    st_ref[1:2, :] += jnp.sum(t * t, axis=0, keepdims=True)


def _k7(t_ref, x_ref, st_ref, g3_ref, b3_ref, o_ref):
    m3 = st_ref[0:1, :] / N
    var3 = st_ref[1:2, :] / N - m3 * m3
    a3 = lax.rsqrt(var3 + EPS) * g3_ref[...]
    o_ref[...] = jnp.maximum(
        (t_ref[...] - m3) * a3 + b3_ref[...] + x_ref[...], 0.0)


def kernel(p, x, o, W1, g1, b1, W2, g2, b2, W3, g3, b3):
    p = _f32(p)
    x = _f32(x)
    f32 = jnp.float32

    row = lambda a: a.reshape(1, D)
    rowspec = pl.BlockSpec((1, D), lambda i: (0, 0))
    full = lambda r, c: pl.BlockSpec((r, c), lambda i: (0, 0))
    blk = pl.BlockSpec((BLK, D), lambda i: (i, 0))
    stspec = pl.BlockSpec((8, D), lambda i: (0, 0))

    h, st1 = pl.pallas_call(
        _k1,
        grid=(N // BLK,),
        in_specs=[blk, full(D, D)],
        out_specs=[blk, stspec],
        out_shape=[jax.ShapeDtypeStruct((N, D), f32),
                   jax.ShapeDtypeStruct((8, D), f32)],
    )(x, W1.T)

    p_pad = jnp.zeros((N, D), f32).at[:, :3].set(p)
    w2xt = jnp.zeros((D, D), f32).at[:3, :].set(W2[:, :3].T)
    u, v = pl.pallas_call(
        _k2,
        grid=(N // BLK,),
        in_specs=[blk, blk, full(D, D), full(D, D), stspec,
                  rowspec, rowspec],
        out_specs=[blk, blk],
        out_shape=[jax.ShapeDtypeStruct((N, D), f32),
                   jax.ShapeDtypeStruct((N, D), f32)],
    )(h, p_pad, W2[:, 3:].T, w2xt, st1, row(g1), row(b1))

    qa = jnp.zeros((NP, 8), f32).at[:N, :3].set(p)
    pta = jnp.zeros((8, NP), f32).at[:3, :N].set(2.0 * p.T)
    pta = pta.at[3, N:].set(PAD_H)
    idx = pl.pallas_call(
        _k3,
        grid=(NP // BQ,),
        in_specs=[pl.BlockSpec((BQ, 8), lambda i: (i, 0)), full(8, NP)],
        out_specs=pl.BlockSpec((BQ, KNN), lambda i: (i, 0)),
        out_shape=jax.ShapeDtypeStruct((NP, KNN), jnp.int32),
    )(qa, pta)

    mx, mn, sm, sq = _gather_reduce(u, idx.reshape(-1))

    sums2 = pl.pallas_call(
        _k5,
        grid=(N // BLK,),
        in_specs=[blk, blk, blk],
        out_specs=stspec,
        out_shape=jax.ShapeDtypeStruct((8, D), f32),
    )(sm[:N], sq[:N], v)

    t, st3 = pl.pallas_call(
        _k6,
        grid=(N // BLK,),
        in_specs=[blk, blk, blk, full(D, D), stspec, rowspec, rowspec],
        out_specs=[blk, stspec],
        out_shape=[jax.ShapeDtypeStruct((N, D), f32),
                   jax.ShapeDtypeStruct((8, D), f32)],
    )(mx[:N], mn[:N], v, W3.T, sums2, row(g2), row(b2))

    out = pl.pallas_call(
        _k7,
        grid=(N // BLK,),
        in_specs=[blk, blk, stspec, rowspec, rowspec],
        out_specs=blk,
        out_shape=jax.ShapeDtypeStruct((N, D), f32),
    )(t, x, st3, row(g3), row(b3))

    return (p, out, o)


# speedup vs baseline: 1.6553x; 1.2837x over previous
import functools

import jax
import jax.numpy as jnp
from jax import lax
from jax.experimental import pallas as pl
from jax.experimental.pallas import tpu as pltpu
from jax.experimental.pallas import tpu_sc as plsc

N = 10000
D = 128
KNN = 16
NP = 10240
EPS = 1e-5
BLK = 1000
BQ = 128
PAD_H = 1e18
NEG = -1e36

NW = 32
NODES_W = NP // NW
SB = 8
EB = SB * KNN


def _f32(x):
    return x.astype(jnp.float32)


def _k1(x_ref, w1t_ref, h_ref, st_ref):
    i = pl.program_id(0)
    h = jnp.dot(x_ref[...], w1t_ref[...], preferred_element_type=jnp.float32)
    h_ref[...] = h

    @pl.when(i == 0)
    def _():
        st_ref[...] = jnp.zeros_like(st_ref)

    st_ref[0:1, :] += jnp.sum(h, axis=0, keepdims=True)
    st_ref[1:2, :] += jnp.sum(h * h, axis=0, keepdims=True)


def _k2(h_ref, p_ref, w2ft_ref, w2xt_ref, st_ref, g1_ref, b1_ref,
        u_ref, v_ref):
    m1 = st_ref[0:1, :] / N
    var1 = st_ref[1:2, :] / N - m1 * m1
    a1 = lax.rsqrt(var1 + EPS) * g1_ref[...]
    hn = jnp.maximum((h_ref[...] - m1) * a1 + b1_ref[...], 0.0)
    v = jnp.dot(p_ref[...], w2xt_ref[...], preferred_element_type=jnp.float32)
    u_ref[...] = jnp.dot(hn, w2ft_ref[...],
                         preferred_element_type=jnp.float32) + v
    v_ref[...] = v


NSEG = 640
GSEG = NP // NSEG


def _k3(qa_ref, pta_ref, idx_ref):
    pta = pta_ref[...]
    pn = 0.25 * jnp.sum(pta * pta, axis=0, keepdims=True)
    score = jnp.dot(qa_ref[...], pta,
                    preferred_element_type=jnp.float32) - pn

    iota = lax.broadcasted_iota(jnp.int32, (BQ, NSEG), 1)

    def sel(t, x, y):
        return (jnp.where(t, x[0], y[0]), jnp.where(t, x[1], y[1]))

    def mrg11(a, b):
        t = a[0] >= b[0]
        return [sel(t, a, b), sel(t, b, a)]

    def mrg22(a, b):
        t = a[0][0] >= b[0][0]
        r1 = sel(t, a[0], b[0])
        x = sel(t, a[1], b[1])
        y = sel(t, b[0], a[0])
        l2 = sel(t, b[1], a[1])
        t2 = x[0] >= y[0]
        r2 = sel(t2, x, y)
        r3 = sel(t2, y, sel(x[0] >= l2[0], x, l2))
        return [r1, r2, r3]

    def mrg33(a, b):
        t = a[0][0] >= b[0][0]
        r1 = sel(t, a[0], b[0])
        x = sel(t, a[1], b[1])
        y = sel(t, b[0], a[0])
        w3 = sel(t, a[2], b[2])
        l2 = sel(t, b[1], a[1])
        t2 = x[0] >= y[0]
        r2 = sel(t2, x, y)
        ca = sel(w3[0] >= y[0], w3, y)
        cb = sel(x[0] >= l2[0], x, l2)
        r3 = sel(t2, ca, cb)
        return [r1, r2, r3]

    lvl = [(score[:, s * NSEG:(s + 1) * NSEG], iota + s * NSEG)
           for s in range(GSEG)]
    pairs = [mrg11(lvl[2 * i], lvl[2 * i + 1]) for i in range(GSEG // 2)]
    tris = [mrg22(pairs[2 * i], pairs[2 * i + 1]) for i in range(GSEG // 4)]
    while len(tris) > 1:
        tris = [mrg33(tris[2 * i], tris[2 * i + 1])
                for i in range(len(tris) // 2)]
    (m1, c1), (m2, c2), (m3, c3) = tris[0]

    cnt = jnp.zeros((BQ, NSEG), jnp.int32)
    cols = []
    for _ in range(KNN):
        m = jnp.max(m1, axis=1, keepdims=True)
        j = jnp.min(jnp.where(m1 == m, iota, NSEG), axis=1, keepdims=True)
        oh = iota == j
        csel = jnp.where(cnt == 0, c1, jnp.where(cnt == 1, c2, c3))
        nxt = jnp.where(cnt == 0, m2, jnp.where(cnt == 1, m3, NEG))
        cols.append(jnp.max(jnp.where(oh, csel, -1), axis=1, keepdims=True))
        m1 = jnp.where(oh, nxt, m1)
        cnt = cnt + oh.astype(jnp.int32)
    idx_ref[...] = jnp.concatenate(cols, axis=1)


def _k4_body(u_hbm, idx_hbm, m_hbm, n_hbm, s_hbm, q_hbm,
             idx_v, rows_v, mv, nv, sv, qv, sem):
    wid = lax.axis_index("s") * 2 + lax.axis_index("c")
    node_base = wid * NODES_W

    def sub_batch(b, carry):
        nb = node_base + b * SB
        pltpu.sync_copy(idx_hbm.at[pl.ds(nb * KNN, EB)], idx_v)
        pltpu.async_copy(u_hbm.at[idx_v], rows_v, sem).wait()

        def node(i, carry2):
            base = i * KNN
            for c in range(D // 16):
                sl = pl.ds(c * 16, 16)
                r0 = rows_v[base, sl]
                am, an, asum, asq = r0, r0, r0, r0 * r0
                for s in range(1, KNN):
                    r = rows_v[base + s, sl]
                    am = jnp.maximum(am, r)
                    an = jnp.minimum(an, r)
                    asum = asum + r
                    asq = asq + r * r
                mv[i, sl] = am
                nv[i, sl] = an
                sv[i, sl] = asum
                qv[i, sl] = asq
            return carry2

        lax.fori_loop(0, SB, node, 0)
        pltpu.sync_copy(mv, m_hbm.at[pl.ds(nb, SB)])
        pltpu.sync_copy(nv, n_hbm.at[pl.ds(nb, SB)])
        pltpu.sync_copy(sv, s_hbm.at[pl.ds(nb, SB)])
        pltpu.sync_copy(qv, q_hbm.at[pl.ds(nb, SB)])
        return carry

    lax.fori_loop(0, NODES_W // SB, sub_batch, 0)


def _gather_reduce(u, idx_flat):
    mesh = plsc.VectorSubcoreMesh(core_axis_name="c", subcore_axis_name="s")
    fn = functools.partial(
        pl.kernel,
        mesh=mesh,
        out_type=[jax.ShapeDtypeStruct((NP, D), jnp.float32)] * 4,
        scratch_types=[
            pltpu.VMEM((EB,), jnp.int32),
            pltpu.VMEM((EB, D), jnp.float32),
            pltpu.VMEM((SB, D), jnp.float32),
            pltpu.VMEM((SB, D), jnp.float32),
            pltpu.VMEM((SB, D), jnp.float32),
            pltpu.VMEM((SB, D), jnp.float32),
            pltpu.SemaphoreType.DMA,
        ],
    )(_k4_body)
    return fn(u, idx_flat)


def _k5(s_ref, q_ref, v_ref, out_ref):
    i = pl.program_id(0)

    @pl.when(i == 0)
    def _():
        out_ref[...] = jnp.zeros_like(out_ref)

    s = s_ref[...]
    v = v_ref[...]
    out_ref[0:1, :] += jnp.sum(s, axis=0, keepdims=True)
    out_ref[1:2, :] += jnp.sum(q_ref[...], axis=0, keepdims=True)
    out_ref[2:3, :] += jnp.sum(v, axis=0, keepdims=True)
    out_ref[3:4, :] += jnp.sum(v * v, axis=0, keepdims=True)
    out_ref[4:5, :] += jnp.sum(v * s, axis=0, keepdims=True)


def _k6(m_ref, n_ref, v_ref, w3t_ref, s2_ref, g2_ref, b2_ref,
        t_ref, st_ref):
    i = pl.program_id(0)
    nk = float(N * KNN)
    s_sum = s2_ref[0:1, :]
    q_sum = s2_ref[1:2, :]
    v_sum = s2_ref[2:3, :]
    v2_sum = s2_ref[3:4, :]
    vs_sum = s2_ref[4:5, :]
    mean2 = (s_sum - KNN * v_sum) / nk
    e2 = (q_sum - 2.0 * vs_sum + KNN * v2_sum) / nk
    var2 = e2 - mean2 * mean2
    a2 = lax.rsqrt(var2 + EPS) * g2_ref[...]
    v = v_ref[...]
    hi = a2 * (m_ref[...] - v - mean2)
    lo = a2 * (n_ref[...] - v - mean2)
    pooled = jnp.maximum(jnp.maximum(hi, lo) + b2_ref[...], 0.0)
    t = jnp.dot(pooled, w3t_ref[...], preferred_element_type=jnp.float32)
    t_ref[...] = t

    @pl.when(i == 0)
    def _():
        st_ref[...] = jnp.zeros_like(st_ref)

    st_ref[0:1, :] += jnp.sum(t, axis=0, keepdims=True)
    st_ref[1:2, :] += jnp.sum(t * t, axis=0, keepdims=True)


def _k7(t_ref, x_ref, st_ref, g3_ref, b3_ref, o_ref):
    m3 = st_ref[0:1, :] / N
    var3 = st_ref[1:2, :] / N - m3 * m3
    a3 = lax.rsqrt(var3 + EPS) * g3_ref[...]
    o_ref[...] = jnp.maximum(
        (t_ref[...] - m3) * a3 + b3_ref[...] + x_ref[...], 0.0)


def kernel(p, x, o, W1, g1, b1, W2, g2, b2, W3, g3, b3):
    p = _f32(p)
    x = _f32(x)
    f32 = jnp.float32

    row = lambda a: a.reshape(1, D)
    rowspec = pl.BlockSpec((1, D), lambda i: (0, 0))
    full = lambda r, c: pl.BlockSpec((r, c), lambda i: (0, 0))
    blk = pl.BlockSpec((BLK, D), lambda i: (i, 0))
    stspec = pl.BlockSpec((8, D), lambda i: (0, 0))

    h, st1 = pl.pallas_call(
        _k1,
        grid=(N // BLK,),
        in_specs=[blk, full(D, D)],
        out_specs=[blk, stspec],
        out_shape=[jax.ShapeDtypeStruct((N, D), f32),
                   jax.ShapeDtypeStruct((8, D), f32)],
    )(x, W1.T)

    p_pad = jnp.zeros((N, D), f32).at[:, :3].set(p)
    w2xt = jnp.zeros((D, D), f32).at[:3, :].set(W2[:, :3].T)
    u, v = pl.pallas_call(
        _k2,
        grid=(N // BLK,),
        in_specs=[blk, blk, full(D, D), full(D, D), stspec,
                  rowspec, rowspec],
        out_specs=[blk, blk],
        out_shape=[jax.ShapeDtypeStruct((N, D), f32),
                   jax.ShapeDtypeStruct((N, D), f32)],
    )(h, p_pad, W2[:, 3:].T, w2xt, st1, row(g1), row(b1))

    qa = jnp.zeros((NP, 8), f32).at[:N, :3].set(p)
    pta = jnp.zeros((8, NP), f32).at[:3, :N].set(2.0 * p.T)
    pta = pta.at[3, N:].set(PAD_H)
    idx = pl.pallas_call(
        _k3,
        grid=(NP // BQ,),
        in_specs=[pl.BlockSpec((BQ, 8), lambda i: (i, 0)), full(8, NP)],
        out_specs=pl.BlockSpec((BQ, KNN), lambda i: (i, 0)),
        out_shape=jax.ShapeDtypeStruct((NP, KNN), jnp.int32),
    )(qa, pta)

    mx, mn, sm, sq = _gather_reduce(u, idx.reshape(-1))

    sums2 = pl.pallas_call(
        _k5,
        grid=(N // BLK,),
        in_specs=[blk, blk, blk],
        out_specs=stspec,
        out_shape=jax.ShapeDtypeStruct((8, D), f32),
    )(sm[:N], sq[:N], v)

    t, st3 = pl.pallas_call(
        _k6,
        grid=(N // BLK,),
        in_specs=[blk, blk, blk, full(D, D), stspec, rowspec, rowspec],
        out_specs=[blk, stspec],
        out_shape=[jax.ShapeDtypeStruct((N, D), f32),
                   jax.ShapeDtypeStruct((8, D), f32)],
    )(mx[:N], mn[:N], v, W3.T, sums2, row(g2), row(b2))

    out = pl.pallas_call(
        _k7,
        grid=(N // BLK,),
        in_specs=[blk, blk, stspec, rowspec, rowspec],
        out_specs=blk,
        out_shape=jax.ShapeDtypeStruct((N, D), f32),
    )(t, x, st3, row(g3), row(b3))

    return (p, out, o)


# speedup vs baseline: 2.0153x; 1.2175x over previous
import functools

import jax
import jax.numpy as jnp
from jax import lax
from jax.experimental import pallas as pl
from jax.experimental.pallas import tpu as pltpu
from jax.experimental.pallas import tpu_sc as plsc

N = 10000
D = 128
KNN = 16
NP = 10240
EPS = 1e-5
BLK = 1000
BQ = 128
PAD_H = 1e18
NEG = -1e36

NW = 32
NODES_W = NP // NW
SB = 8
EB = SB * KNN


def _f32(x):
    return x.astype(jnp.float32)


def _k1(x_ref, w1t_ref, h_ref, st_ref):
    i = pl.program_id(0)
    h = jnp.dot(x_ref[...], w1t_ref[...], preferred_element_type=jnp.float32)
    h_ref[...] = h

    @pl.when(i == 0)
    def _():
        st_ref[...] = jnp.zeros_like(st_ref)

    st_ref[0:1, :] += jnp.sum(h, axis=0, keepdims=True)
    st_ref[1:2, :] += jnp.sum(h * h, axis=0, keepdims=True)


def _k2(h_ref, p_ref, w2ft_ref, w2xt_ref, st_ref, g1_ref, b1_ref,
        u_ref, v_ref):
    m1 = st_ref[0:1, :] / N
    var1 = st_ref[1:2, :] / N - m1 * m1
    a1 = lax.rsqrt(var1 + EPS) * g1_ref[...]
    hn = jnp.maximum((h_ref[...] - m1) * a1 + b1_ref[...], 0.0)
    v = jnp.dot(p_ref[...], w2xt_ref[...], preferred_element_type=jnp.float32)
    u_ref[...] = jnp.dot(hn, w2ft_ref[...],
                         preferred_element_type=jnp.float32) + v
    v_ref[...] = v


NSEG = 640
GSEG = NP // NSEG


def _k3(qa_ref, pta_ref, idx_ref):
    pta = pta_ref[...]
    pn = 0.25 * jnp.sum(pta * pta, axis=0, keepdims=True)
    score = jnp.dot(qa_ref[...], pta,
                    preferred_element_type=jnp.float32) - pn

    iota = lax.broadcasted_iota(jnp.int32, (BQ, NSEG), 1)

    def sel(t, x, y):
        return (jnp.where(t, x[0], y[0]), jnp.where(t, x[1], y[1]))

    def mrg11(a, b):
        t = a[0] >= b[0]
        return [sel(t, a, b), sel(t, b, a)]

    def mrg22(a, b):
        t = a[0][0] >= b[0][0]
        r1 = sel(t, a[0], b[0])
        x = sel(t, a[1], b[1])
        y = sel(t, b[0], a[0])
        l2 = sel(t, b[1], a[1])
        t2 = x[0] >= y[0]
        r2 = sel(t2, x, y)
        r3 = sel(t2, y, sel(x[0] >= l2[0], x, l2))
        return [r1, r2, r3]

    def mrg33(a, b):
        t = a[0][0] >= b[0][0]
        r1 = sel(t, a[0], b[0])
        x = sel(t, a[1], b[1])
        y = sel(t, b[0], a[0])
        w3 = sel(t, a[2], b[2])
        l2 = sel(t, b[1], a[1])
        t2 = x[0] >= y[0]
        r2 = sel(t2, x, y)
        ca = sel(w3[0] >= y[0], w3, y)
        cb = sel(x[0] >= l2[0], x, l2)
        r3 = sel(t2, ca, cb)
        return [r1, r2, r3]

    lvl = [(score[:, s * NSEG:(s + 1) * NSEG], iota + s * NSEG)
           for s in range(GSEG)]
    pairs = [mrg11(lvl[2 * i], lvl[2 * i + 1]) for i in range(GSEG // 2)]
    tris = [mrg22(pairs[2 * i], pairs[2 * i + 1]) for i in range(GSEG // 4)]
    while len(tris) > 1:
        tris = [mrg33(tris[2 * i], tris[2 * i + 1])
                for i in range(len(tris) // 2)]
    (m1, c1), (m2, c2), (m3, c3) = tris[0]

    cols = []
    for _ in range(KNN):
        m = jnp.max(m1, axis=1, keepdims=True)
        oh = m1 == m
        cols.append(jnp.max(jnp.where(oh, c1, -1), axis=1, keepdims=True))
        m1 = jnp.where(oh, m2, m1)
        c1 = jnp.where(oh, c2, c1)
        m2 = jnp.where(oh, m3, m2)
        c2 = jnp.where(oh, c3, c2)
        m3 = jnp.where(oh, NEG, m3)
    idx_ref[...] = jnp.concatenate(cols, axis=1)


def _k4_body(u_hbm, idx_hbm, m_hbm, n_hbm, s_hbm, q_hbm,
             idx_v, rows_v, mv, nv, sv, qv, sem):
    wid = lax.axis_index("s") * 2 + lax.axis_index("c")
    node_base = wid * NODES_W

    def sub_batch(b, carry):
        nb = node_base + b * SB
        pltpu.sync_copy(idx_hbm.at[pl.ds(nb * KNN, EB)], idx_v)
        pltpu.async_copy(u_hbm.at[idx_v], rows_v, sem).wait()

        def node(i, carry2):
            base = i * KNN
            for c in range(D // 16):
                sl = pl.ds(c * 16, 16)
                r0 = rows_v[base, sl]
                am, an, asum, asq = r0, r0, r0, r0 * r0
                for s in range(1, KNN):
                    r = rows_v[base + s, sl]
                    am = jnp.maximum(am, r)
                    an = jnp.minimum(an, r)
                    asum = asum + r
                    asq = asq + r * r
                mv[i, sl] = am
                nv[i, sl] = an
                sv[i, sl] = asum
                qv[i, sl] = asq
            return carry2

        lax.fori_loop(0, SB, node, 0)
        pltpu.sync_copy(mv, m_hbm.at[pl.ds(nb, SB)])
        pltpu.sync_copy(nv, n_hbm.at[pl.ds(nb, SB)])
        pltpu.sync_copy(sv, s_hbm.at[pl.ds(nb, SB)])
        pltpu.sync_copy(qv, q_hbm.at[pl.ds(nb, SB)])
        return carry

    lax.fori_loop(0, NODES_W // SB, sub_batch, 0)


def _gather_reduce(u, idx_flat):
    mesh = plsc.VectorSubcoreMesh(core_axis_name="c", subcore_axis_name="s")
    fn = functools.partial(
        pl.kernel,
        mesh=mesh,
        out_type=[jax.ShapeDtypeStruct((NP, D), jnp.float32)] * 4,
        scratch_types=[
            pltpu.VMEM((EB,), jnp.int32),
            pltpu.VMEM((EB, D), jnp.float32),
            pltpu.VMEM((SB, D), jnp.float32),
            pltpu.VMEM((SB, D), jnp.float32),
            pltpu.VMEM((SB, D), jnp.float32),
            pltpu.VMEM((SB, D), jnp.float32),
            pltpu.SemaphoreType.DMA,
        ],
    )(_k4_body)
    return fn(u, idx_flat)


def _k5(s_ref, q_ref, v_ref, out_ref):
    i = pl.program_id(0)

    @pl.when(i == 0)
    def _():
        out_ref[...] = jnp.zeros_like(out_ref)

    s = s_ref[...]
    v = v_ref[...]
    out_ref[0:1, :] += jnp.sum(s, axis=0, keepdims=True)
    out_ref[1:2, :] += jnp.sum(q_ref[...], axis=0, keepdims=True)
    out_ref[2:3, :] += jnp.sum(v, axis=0, keepdims=True)
    out_ref[3:4, :] += jnp.sum(v * v, axis=0, keepdims=True)
    out_ref[4:5, :] += jnp.sum(v * s, axis=0, keepdims=True)


def _k6(m_ref, n_ref, v_ref, w3t_ref, s2_ref, g2_ref, b2_ref,
        t_ref, st_ref):
    i = pl.program_id(0)
    nk = float(N * KNN)
    s_sum = s2_ref[0:1, :]
    q_sum = s2_ref[1:2, :]
    v_sum = s2_ref[2:3, :]
    v2_sum = s2_ref[3:4, :]
    vs_sum = s2_ref[4:5, :]
    mean2 = (s_sum - KNN * v_sum) / nk
    e2 = (q_sum - 2.0 * vs_sum + KNN * v2_sum) / nk
    var2 = e2 - mean2 * mean2
    a2 = lax.rsqrt(var2 + EPS) * g2_ref[...]
    v = v_ref[...]
    hi = a2 * (m_ref[...] - v - mean2)
    lo = a2 * (n_ref[...] - v - mean2)
    pooled = jnp.maximum(jnp.maximum(hi, lo) + b2_ref[...], 0.0)
    t = jnp.dot(pooled, w3t_ref[...], preferred_element_type=jnp.float32)
    t_ref[...] = t

    @pl.when(i == 0)
    def _():
        st_ref[...] = jnp.zeros_like(st_ref)

    st_ref[0:1, :] += jnp.sum(t, axis=0, keepdims=True)
    st_ref[1:2, :] += jnp.sum(t * t, axis=0, keepdims=True)


def _k7(t_ref, x_ref, st_ref, g3_ref, b3_ref, o_ref):
    m3 = st_ref[0:1, :] / N
    var3 = st_ref[1:2, :] / N - m3 * m3
    a3 = lax.rsqrt(var3 + EPS) * g3_ref[...]
    o_ref[...] = jnp.maximum(
        (t_ref[...] - m3) * a3 + b3_ref[...] + x_ref[...], 0.0)


def kernel(p, x, o, W1, g1, b1, W2, g2, b2, W3, g3, b3):
    p = _f32(p)
    x = _f32(x)
    f32 = jnp.float32

    row = lambda a: a.reshape(1, D)
    rowspec = pl.BlockSpec((1, D), lambda i: (0, 0))
    full = lambda r, c: pl.BlockSpec((r, c), lambda i: (0, 0))
    blk = pl.BlockSpec((BLK, D), lambda i: (i, 0))
    stspec = pl.BlockSpec((8, D), lambda i: (0, 0))

    h, st1 = pl.pallas_call(
        _k1,
        grid=(N // BLK,),
        in_specs=[blk, full(D, D)],
        out_specs=[blk, stspec],
        out_shape=[jax.ShapeDtypeStruct((N, D), f32),
                   jax.ShapeDtypeStruct((8, D), f32)],
    )(x, W1.T)

    p_pad = jnp.zeros((N, D), f32).at[:, :3].set(p)
    w2xt = jnp.zeros((D, D), f32).at[:3, :].set(W2[:, :3].T)
    u, v = pl.pallas_call(
        _k2,
        grid=(N // BLK,),
        in_specs=[blk, blk, full(D, D), full(D, D), stspec,
                  rowspec, rowspec],
        out_specs=[blk, blk],
        out_shape=[jax.ShapeDtypeStruct((N, D), f32),
                   jax.ShapeDtypeStruct((N, D), f32)],
    )(h, p_pad, W2[:, 3:].T, w2xt, st1, row(g1), row(b1))

    qa = jnp.zeros((NP, 8), f32).at[:N, :3].set(p)
    pta = jnp.zeros((8, NP), f32).at[:3, :N].set(2.0 * p.T)
    pta = pta.at[3, N:].set(PAD_H)
    idx = pl.pallas_call(
        _k3,
        grid=(NP // BQ,),
        in_specs=[pl.BlockSpec((BQ, 8), lambda i: (i, 0)), full(8, NP)],
        out_specs=pl.BlockSpec((BQ, KNN), lambda i: (i, 0)),
        out_shape=jax.ShapeDtypeStruct((NP, KNN), jnp.int32),
    )(qa, pta)

    mx, mn, sm, sq = _gather_reduce(u, idx.reshape(-1))

    sums2 = pl.pallas_call(
        _k5,
        grid=(N // BLK,),
        in_specs=[blk, blk, blk],
        out_specs=stspec,
        out_shape=jax.ShapeDtypeStruct((8, D), f32),
    )(sm[:N], sq[:N], v)

    t, st3 = pl.pallas_call(
        _k6,
        grid=(N // BLK,),
        in_specs=[blk, blk, blk, full(D, D), stspec, rowspec, rowspec],
        out_specs=[blk, stspec],
        out_shape=[jax.ShapeDtypeStruct((N, D), f32),
                   jax.ShapeDtypeStruct((8, D), f32)],
    )(mx[:N], mn[:N], v, W3.T, sums2, row(g2), row(b2))

    out = pl.pallas_call(
        _k7,
        grid=(N // BLK,),
        in_specs=[blk, blk, stspec, rowspec, rowspec],
        out_specs=blk,
        out_shape=jax.ShapeDtypeStruct((N, D), f32),
    )(t, x, st3, row(g3), row(b3))

    return (p, out, o)


# speedup vs baseline: 2.1797x; 1.0815x over previous
import functools

import jax
import jax.numpy as jnp
from jax import lax
from jax.experimental import pallas as pl
from jax.experimental.pallas import tpu as pltpu
from jax.experimental.pallas import tpu_sc as plsc

N = 10000
D = 128
KNN = 16
NP = 10240
EPS = 1e-5
BLK = 1000
BQ = 128
PAD_H = 1e18
NEG = -1e36

NW = 32
NODES_W = NP // NW
SB = 8
EB = SB * KNN


def _f32(x):
    return x.astype(jnp.float32)


def _k1(x_ref, w1t_ref, h_ref, st_ref):
    i = pl.program_id(0)
    h = jnp.dot(x_ref[...], w1t_ref[...], preferred_element_type=jnp.float32)
    h_ref[...] = h

    @pl.when(i == 0)
    def _():
        st_ref[...] = jnp.zeros_like(st_ref)

    st_ref[0:1, :] += jnp.sum(h, axis=0, keepdims=True)
    st_ref[1:2, :] += jnp.sum(h * h, axis=0, keepdims=True)


def _k2(h_ref, p_ref, w2ft_ref, w2xt_ref, st_ref, g1_ref, b1_ref,
        u_ref, v_ref):
    m1 = st_ref[0:1, :] / N
    var1 = st_ref[1:2, :] / N - m1 * m1
    a1 = lax.rsqrt(var1 + EPS) * g1_ref[...]
    hn = jnp.maximum((h_ref[...] - m1) * a1 + b1_ref[...], 0.0)
    v = jnp.dot(p_ref[...], w2xt_ref[...], preferred_element_type=jnp.float32)
    u_ref[...] = jnp.dot(hn, w2ft_ref[...],
                         preferred_element_type=jnp.float32) + v
    v_ref[...] = v


NSEG = 640
GSEG = NP // NSEG


def _k3(qa_ref, pta_ref, idx_ref):
    pta = pta_ref[...]
    pn = 0.25 * jnp.sum(pta * pta, axis=0, keepdims=True)
    score = jnp.dot(qa_ref[...], pta,
                    preferred_element_type=jnp.float32) - pn

    iota = lax.broadcasted_iota(jnp.int32, (BQ, NSEG), 1)

    def sel(t, x, y):
        return (jnp.where(t, x[0], y[0]), jnp.where(t, x[1], y[1]))

    def mrg11(a, b):
        t = a[0] >= b[0]
        return [sel(t, a, b), sel(t, b, a)]

    def mrg22(a, b):
        t = a[0][0] >= b[0][0]
        r1 = sel(t, a[0], b[0])
        x = sel(t, a[1], b[1])
        y = sel(t, b[0], a[0])
        l2 = sel(t, b[1], a[1])
        t2 = x[0] >= y[0]
        r2 = sel(t2, x, y)
        r3 = sel(t2, y, sel(x[0] >= l2[0], x, l2))
        return [r1, r2, r3]

    def mrg33(a, b):
        t = a[0][0] >= b[0][0]
        r1 = sel(t, a[0], b[0])
        x = sel(t, a[1], b[1])
        y = sel(t, b[0], a[0])
        w3 = sel(t, a[2], b[2])
        l2 = sel(t, b[1], a[1])
        t2 = x[0] >= y[0]
        r2 = sel(t2, x, y)
        ca = sel(w3[0] >= y[0], w3, y)
        cb = sel(x[0] >= l2[0], x, l2)
        r3 = sel(t2, ca, cb)
        return [r1, r2, r3]

    lvl = [(score[:, s * NSEG:(s + 1) * NSEG], iota + s * NSEG)
           for s in range(GSEG)]
    pairs = [mrg11(lvl[2 * i], lvl[2 * i + 1]) for i in range(GSEG // 2)]
    tris = [mrg22(pairs[2 * i], pairs[2 * i + 1]) for i in range(GSEG // 4)]
    while len(tris) > 1:
        tris = [mrg33(tris[2 * i], tris[2 * i + 1])
                for i in range(len(tris) // 2)]
    (m1, c1), (m2, c2), (m3, c3) = tris[0]

    cols = []
    for _ in range(KNN):
        m = jnp.max(m1, axis=1, keepdims=True)
        oh = m1 == m
        cols.append(jnp.max(jnp.where(oh, c1, -1), axis=1, keepdims=True))
        m1 = jnp.where(oh, m2, m1)
        c1 = jnp.where(oh, c2, c1)
        m2 = jnp.where(oh, m3, m2)
        c2 = jnp.where(oh, c3, c2)
        m3 = jnp.where(oh, NEG, m3)
    idx_ref[...] = jnp.concatenate(cols, axis=1)


NB = NODES_W // SB


def _k4_body(u_hbm, idx_hbm, m_hbm, n_hbm, s_hbm, q_hbm,
             idx_v0, rows_v0, idx_v1, rows_v1, mv, nv, sv, qv,
             sem0, sem1):
    wid = lax.axis_index("s") * 2 + lax.axis_index("c")
    node_base = wid * NODES_W
    bufs = ((idx_v0, rows_v0, sem0), (idx_v1, rows_v1, sem1))

    def start(b, buf):
        idx_v, rows_v, sem = buf
        pltpu.sync_copy(idx_hbm.at[pl.ds((node_base + b * SB) * KNN, EB)],
                        idx_v)
        pltpu.async_copy(u_hbm.at[idx_v], rows_v, sem)

    def process(b, cur, nxt):
        idx_v, rows_v, sem = cur

        @pl.when(b + 1 < NB)
        def _():
            start(b + 1, nxt)

        pltpu.make_async_copy(u_hbm.at[idx_v], rows_v, sem).wait()

        def node(i, carry2):
            base = i * KNN
            for c in range(D // 16):
                sl = pl.ds(c * 16, 16)
                r0 = rows_v[base, sl]
                am, an, asum, asq = r0, r0, r0, r0 * r0
                for s in range(1, KNN):
                    r = rows_v[base + s, sl]
                    am = jnp.maximum(am, r)
                    an = jnp.minimum(an, r)
                    asum = asum + r
                    asq = asq + r * r
                mv[i, sl] = am
                nv[i, sl] = an
                sv[i, sl] = asum
                qv[i, sl] = asq
            return carry2

        lax.fori_loop(0, SB, node, 0)
        nb = node_base + b * SB
        pltpu.sync_copy(mv, m_hbm.at[pl.ds(nb, SB)])
        pltpu.sync_copy(nv, n_hbm.at[pl.ds(nb, SB)])
        pltpu.sync_copy(sv, s_hbm.at[pl.ds(nb, SB)])
        pltpu.sync_copy(qv, q_hbm.at[pl.ds(nb, SB)])

    start(0, bufs[0])

    def pair(i, carry):
        process(2 * i, bufs[0], bufs[1])
        process(2 * i + 1, bufs[1], bufs[0])
        return carry

    lax.fori_loop(0, NB // 2, pair, 0)


def _gather_reduce(u, idx_flat):
    mesh = plsc.VectorSubcoreMesh(core_axis_name="c", subcore_axis_name="s")
    fn = functools.partial(
        pl.kernel,
        mesh=mesh,
        out_type=[jax.ShapeDtypeStruct((NP, D), jnp.float32)] * 4,
        scratch_types=[
            pltpu.VMEM((EB,), jnp.int32),
            pltpu.VMEM((EB, D), jnp.float32),
            pltpu.VMEM((EB,), jnp.int32),
            pltpu.VMEM((EB, D), jnp.float32),
            pltpu.VMEM((SB, D), jnp.float32),
            pltpu.VMEM((SB, D), jnp.float32),
            pltpu.VMEM((SB, D), jnp.float32),
            pltpu.VMEM((SB, D), jnp.float32),
            pltpu.SemaphoreType.DMA,
            pltpu.SemaphoreType.DMA,
        ],
    )(_k4_body)
    return fn(u, idx_flat)


def _k5(s_ref, q_ref, v_ref, out_ref):
    i = pl.program_id(0)

    @pl.when(i == 0)
    def _():
        out_ref[...] = jnp.zeros_like(out_ref)

    s = s_ref[...]
    v = v_ref[...]
    out_ref[0:1, :] += jnp.sum(s, axis=0, keepdims=True)
    out_ref[1:2, :] += jnp.sum(q_ref[...], axis=0, keepdims=True)
    out_ref[2:3, :] += jnp.sum(v, axis=0, keepdims=True)
    out_ref[3:4, :] += jnp.sum(v * v, axis=0, keepdims=True)
    out_ref[4:5, :] += jnp.sum(v * s, axis=0, keepdims=True)


def _k6(m_ref, n_ref, v_ref, w3t_ref, s2_ref, g2_ref, b2_ref,
        t_ref, st_ref):
    i = pl.program_id(0)
    nk = float(N * KNN)
    s_sum = s2_ref[0:1, :]
    q_sum = s2_ref[1:2, :]
    v_sum = s2_ref[2:3, :]
    v2_sum = s2_ref[3:4, :]
    vs_sum = s2_ref[4:5, :]
    mean2 = (s_sum - KNN * v_sum) / nk
    e2 = (q_sum - 2.0 * vs_sum + KNN * v2_sum) / nk
    var2 = e2 - mean2 * mean2
    a2 = lax.rsqrt(var2 + EPS) * g2_ref[...]
    v = v_ref[...]
    hi = a2 * (m_ref[...] - v - mean2)
    lo = a2 * (n_ref[...] - v - mean2)
    pooled = jnp.maximum(jnp.maximum(hi, lo) + b2_ref[...], 0.0)
    t = jnp.dot(pooled, w3t_ref[...], preferred_element_type=jnp.float32)
    t_ref[...] = t

    @pl.when(i == 0)
    def _():
        st_ref[...] = jnp.zeros_like(st_ref)

    st_ref[0:1, :] += jnp.sum(t, axis=0, keepdims=True)
    st_ref[1:2, :] += jnp.sum(t * t, axis=0, keepdims=True)


def _k7(t_ref, x_ref, st_ref, g3_ref, b3_ref, o_ref):
    m3 = st_ref[0:1, :] / N
    var3 = st_ref[1:2, :] / N - m3 * m3
    a3 = lax.rsqrt(var3 + EPS) * g3_ref[...]
    o_ref[...] = jnp.maximum(
        (t_ref[...] - m3) * a3 + b3_ref[...] + x_ref[...], 0.0)


def kernel(p, x, o, W1, g1, b1, W2, g2, b2, W3, g3, b3):
    p = _f32(p)
    x = _f32(x)
    f32 = jnp.float32

    row = lambda a: a.reshape(1, D)
    rowspec = pl.BlockSpec((1, D), lambda i: (0, 0))
    full = lambda r, c: pl.BlockSpec((r, c), lambda i: (0, 0))
    blk = pl.BlockSpec((BLK, D), lambda i: (i, 0))
    stspec = pl.BlockSpec((8, D), lambda i: (0, 0))

    h, st1 = pl.pallas_call(
        _k1,
        grid=(N // BLK,),
        in_specs=[blk, full(D, D)],
        out_specs=[blk, stspec],
        out_shape=[jax.ShapeDtypeStruct((N, D), f32),
                   jax.ShapeDtypeStruct((8, D), f32)],
    )(x, W1.T)

    p_pad = jnp.zeros((N, D), f32).at[:, :3].set(p)
    w2xt = jnp.zeros((D, D), f32).at[:3, :].set(W2[:, :3].T)
    u, v = pl.pallas_call(
        _k2,
        grid=(N // BLK,),
        in_specs=[blk, blk, full(D, D), full(D, D), stspec,
                  rowspec, rowspec],
        out_specs=[blk, blk],
        out_shape=[jax.ShapeDtypeStruct((N, D), f32),
                   jax.ShapeDtypeStruct((N, D), f32)],
    )(h, p_pad, W2[:, 3:].T, w2xt, st1, row(g1), row(b1))

    qa = jnp.zeros((NP, 8), f32).at[:N, :3].set(p)
    pta = jnp.zeros((8, NP), f32).at[:3, :N].set(2.0 * p.T)
    pta = pta.at[3, N:].set(PAD_H)
    idx = pl.pallas_call(
        _k3,
        grid=(NP // BQ,),
        in_specs=[pl.BlockSpec((BQ, 8), lambda i: (i, 0)), full(8, NP)],
        out_specs=pl.BlockSpec((BQ, KNN), lambda i: (i, 0)),
        out_shape=jax.ShapeDtypeStruct((NP, KNN), jnp.int32),
    )(qa, pta)

    mx, mn, sm, sq = _gather_reduce(u, idx.reshape(-1))

    sums2 = pl.pallas_call(
        _k5,
        grid=(N // BLK,),
        in_specs=[blk, blk, blk],
        out_specs=stspec,
        out_shape=jax.ShapeDtypeStruct((8, D), f32),
    )(sm[:N], sq[:N], v)

    t, st3 = pl.pallas_call(
        _k6,
        grid=(N // BLK,),
        in_specs=[blk, blk, blk, full(D, D), stspec, rowspec, rowspec],
        out_specs=[blk, stspec],
        out_shape=[jax.ShapeDtypeStruct((N, D), f32),
                   jax.ShapeDtypeStruct((8, D), f32)],
    )(mx[:N], mn[:N], v, W3.T, sums2, row(g2), row(b2))

    out = pl.pallas_call(
        _k7,
        grid=(N // BLK,),
        in_specs=[blk, blk, stspec, rowspec, rowspec],
        out_specs=blk,
        out_shape=jax.ShapeDtypeStruct((N, D), f32),
    )(t, x, st3, row(g3), row(b3))

    return (p, out, o)


# speedup vs baseline: 2.1845x; 1.0022x over previous
import functools

import jax
import jax.numpy as jnp
from jax import lax
from jax.experimental import pallas as pl
from jax.experimental.pallas import tpu as pltpu
from jax.experimental.pallas import tpu_sc as plsc

N = 10000
D = 128
KNN = 16
NP = 10240
EPS = 1e-5
BLK = 1000
BQ = 256
PAD_H = 1e18
NEG = -1e36

NW = 32
NODES_W = NP // NW
SB = 8
EB = SB * KNN


def _f32(x):
    return x.astype(jnp.float32)


def _k1(x_ref, w1t_ref, h_ref, st_ref):
    i = pl.program_id(0)
    h = jnp.dot(x_ref[...], w1t_ref[...], preferred_element_type=jnp.float32)
    h_ref[...] = h

    @pl.when(i == 0)
    def _():
        st_ref[...] = jnp.zeros_like(st_ref)

    st_ref[0:1, :] += jnp.sum(h, axis=0, keepdims=True)
    st_ref[1:2, :] += jnp.sum(h * h, axis=0, keepdims=True)


def _k2(h_ref, p_ref, w2ft_ref, w2xt_ref, st_ref, g1_ref, b1_ref,
        u_ref, v_ref):
    m1 = st_ref[0:1, :] / N
    var1 = st_ref[1:2, :] / N - m1 * m1
    a1 = lax.rsqrt(var1 + EPS) * g1_ref[...]
    hn = jnp.maximum((h_ref[...] - m1) * a1 + b1_ref[...], 0.0)
    v = jnp.dot(p_ref[...], w2xt_ref[...], preferred_element_type=jnp.float32)
    u_ref[...] = jnp.dot(hn, w2ft_ref[...],
                         preferred_element_type=jnp.float32) + v
    v_ref[...] = v


NSEG = 640
GSEG = NP // NSEG


def _k3(qa_ref, pta_ref, idx_ref):
    pta = pta_ref[...]
    pn = 0.25 * jnp.sum(pta * pta, axis=0, keepdims=True)
    score = jnp.dot(qa_ref[...], pta,
                    preferred_element_type=jnp.float32) - pn

    iota = lax.broadcasted_iota(jnp.int32, (BQ, NSEG), 1)

    def sel(t, x, y):
        return (jnp.where(t, x[0], y[0]), jnp.where(t, x[1], y[1]))

    def mrg11(a, b):
        t = a[0] >= b[0]
        return [sel(t, a, b), sel(t, b, a)]

    def mrg22(a, b):
        t = a[0][0] >= b[0][0]
        r1 = sel(t, a[0], b[0])
        x = sel(t, a[1], b[1])
        y = sel(t, b[0], a[0])
        l2 = sel(t, b[1], a[1])
        t2 = x[0] >= y[0]
        r2 = sel(t2, x, y)
        r3 = sel(t2, y, sel(x[0] >= l2[0], x, l2))
        return [r1, r2, r3]

    def mrg33(a, b):
        t = a[0][0] >= b[0][0]
        r1 = sel(t, a[0], b[0])
        x = sel(t, a[1], b[1])
        y = sel(t, b[0], a[0])
        w3 = sel(t, a[2], b[2])
        l2 = sel(t, b[1], a[1])
        t2 = x[0] >= y[0]
        r2 = sel(t2, x, y)
        ca = sel(w3[0] >= y[0], w3, y)
        cb = sel(x[0] >= l2[0], x, l2)
        r3 = sel(t2, ca, cb)
        return [r1, r2, r3]

    lvl = [(score[:, s * NSEG:(s + 1) * NSEG], iota + s * NSEG)
           for s in range(GSEG)]
    pairs = [mrg11(lvl[2 * i], lvl[2 * i + 1]) for i in range(GSEG // 2)]
    tris = [mrg22(pairs[2 * i], pairs[2 * i + 1]) for i in range(GSEG // 4)]
    while len(tris) > 1:
        tris = [mrg33(tris[2 * i], tris[2 * i + 1])
                for i in range(len(tris) // 2)]
    (m1, c1), (m2, c2), (m3, c3) = tris[0]

    cols = []
    for _ in range(KNN):
        m = jnp.max(m1, axis=1, keepdims=True)
        oh = m1 == m
        cols.append(jnp.max(jnp.where(oh, c1, -1), axis=1, keepdims=True))
        m1 = jnp.where(oh, m2, m1)
        c1 = jnp.where(oh, c2, c1)
        m2 = jnp.where(oh, m3, m2)
        c2 = jnp.where(oh, c3, c2)
        m3 = jnp.where(oh, NEG, m3)
    idx_ref[...] = jnp.concatenate(cols, axis=1)


NB = NODES_W // SB


def _k4_body(u_hbm, idx_hbm, m_hbm, n_hbm, s_hbm, q_hbm,
             idx_v0, rows_v0, idx_v1, rows_v1, mv, nv, sv, qv,
             sem0, sem1):
    wid = lax.axis_index("s") * 2 + lax.axis_index("c")
    node_base = wid * NODES_W
    bufs = ((idx_v0, rows_v0, sem0), (idx_v1, rows_v1, sem1))

    def start(b, buf):
        idx_v, rows_v, sem = buf
        pltpu.sync_copy(idx_hbm.at[pl.ds((node_base + b * SB) * KNN, EB)],
                        idx_v)
        pltpu.async_copy(u_hbm.at[idx_v], rows_v, sem)

    def process(b, cur, nxt):
        idx_v, rows_v, sem = cur

        @pl.when(b + 1 < NB)
        def _():
            start(b + 1, nxt)

        pltpu.make_async_copy(u_hbm.at[idx_v], rows_v, sem).wait()

        def node(i, carry2):
            base = i * KNN
            for c in range(D // 16):
                sl = pl.ds(c * 16, 16)
                r0 = rows_v[base, sl]
                am, an, asum, asq = r0, r0, r0, r0 * r0
                for s in range(1, KNN):
                    r = rows_v[base + s, sl]
                    am = jnp.maximum(am, r)
                    an = jnp.minimum(an, r)
                    asum = asum + r
                    asq = asq + r * r
                mv[i, sl] = am
                nv[i, sl] = an
                sv[i, sl] = asum
                qv[i, sl] = asq
            return carry2

        lax.fori_loop(0, SB, node, 0)
        nb = node_base + b * SB
        pltpu.sync_copy(mv, m_hbm.at[pl.ds(nb, SB)])
        pltpu.sync_copy(nv, n_hbm.at[pl.ds(nb, SB)])
        pltpu.sync_copy(sv, s_hbm.at[pl.ds(nb, SB)])
        pltpu.sync_copy(qv, q_hbm.at[pl.ds(nb, SB)])

    start(0, bufs[0])

    def pair(i, carry):
        process(2 * i, bufs[0], bufs[1])
        process(2 * i + 1, bufs[1], bufs[0])
        return carry

    lax.fori_loop(0, NB // 2, pair, 0)


def _gather_reduce(u, idx_flat):
    mesh = plsc.VectorSubcoreMesh(core_axis_name="c", subcore_axis_name="s")
    fn = functools.partial(
        pl.kernel,
        mesh=mesh,
        out_type=[jax.ShapeDtypeStruct((NP, D), jnp.float32)] * 4,
        scratch_types=[
            pltpu.VMEM((EB,), jnp.int32),
            pltpu.VMEM((EB, D), jnp.float32),
            pltpu.VMEM((EB,), jnp.int32),
            pltpu.VMEM((EB, D), jnp.float32),
            pltpu.VMEM((SB, D), jnp.float32),
            pltpu.VMEM((SB, D), jnp.float32),
            pltpu.VMEM((SB, D), jnp.float32),
            pltpu.VMEM((SB, D), jnp.float32),
            pltpu.SemaphoreType.DMA,
            pltpu.SemaphoreType.DMA,
        ],
    )(_k4_body)
    return fn(u, idx_flat)


def _k5(s_ref, q_ref, v_ref, out_ref):
    i = pl.program_id(0)

    @pl.when(i == 0)
    def _():
        out_ref[...] = jnp.zeros_like(out_ref)

    s = s_ref[...]
    v = v_ref[...]
    out_ref[0:1, :] += jnp.sum(s, axis=0, keepdims=True)
    out_ref[1:2, :] += jnp.sum(q_ref[...], axis=0, keepdims=True)
    out_ref[2:3, :] += jnp.sum(v, axis=0, keepdims=True)
    out_ref[3:4, :] += jnp.sum(v * v, axis=0, keepdims=True)
    out_ref[4:5, :] += jnp.sum(v * s, axis=0, keepdims=True)


def _k6(m_ref, n_ref, v_ref, w3t_ref, s2_ref, g2_ref, b2_ref,
        t_ref, st_ref):
    i = pl.program_id(0)
    nk = float(N * KNN)
    s_sum = s2_ref[0:1, :]
    q_sum = s2_ref[1:2, :]
    v_sum = s2_ref[2:3, :]
    v2_sum = s2_ref[3:4, :]
    vs_sum = s2_ref[4:5, :]
    mean2 = (s_sum - KNN * v_sum) / nk
    e2 = (q_sum - 2.0 * vs_sum + KNN * v2_sum) / nk
    var2 = e2 - mean2 * mean2
    a2 = lax.rsqrt(var2 + EPS) * g2_ref[...]
    v = v_ref[...]
    hi = a2 * (m_ref[...] - v - mean2)
    lo = a2 * (n_ref[...] - v - mean2)
    pooled = jnp.maximum(jnp.maximum(hi, lo) + b2_ref[...], 0.0)
    t = jnp.dot(pooled, w3t_ref[...], preferred_element_type=jnp.float32)
    t_ref[...] = t

    @pl.when(i == 0)
    def _():
        st_ref[...] = jnp.zeros_like(st_ref)

    st_ref[0:1, :] += jnp.sum(t, axis=0, keepdims=True)
    st_ref[1:2, :] += jnp.sum(t * t, axis=0, keepdims=True)


def _k7(t_ref, x_ref, st_ref, g3_ref, b3_ref, o_ref):
    m3 = st_ref[0:1, :] / N
    var3 = st_ref[1:2, :] / N - m3 * m3
    a3 = lax.rsqrt(var3 + EPS) * g3_ref[...]
    o_ref[...] = jnp.maximum(
        (t_ref[...] - m3) * a3 + b3_ref[...] + x_ref[...], 0.0)


def kernel(p, x, o, W1, g1, b1, W2, g2, b2, W3, g3, b3):
    p = _f32(p)
    x = _f32(x)
    f32 = jnp.float32

    row = lambda a: a.reshape(1, D)
    rowspec = pl.BlockSpec((1, D), lambda i: (0, 0))
    full = lambda r, c: pl.BlockSpec((r, c), lambda i: (0, 0))
    blk = pl.BlockSpec((BLK, D), lambda i: (i, 0))
    stspec = pl.BlockSpec((8, D), lambda i: (0, 0))

    h, st1 = pl.pallas_call(
        _k1,
        grid=(N // BLK,),
        in_specs=[blk, full(D, D)],
        out_specs=[blk, stspec],
        out_shape=[jax.ShapeDtypeStruct((N, D), f32),
                   jax.ShapeDtypeStruct((8, D), f32)],
    )(x, W1.T)

    p_pad = jnp.zeros((N, D), f32).at[:, :3].set(p)
    w2xt = jnp.zeros((D, D), f32).at[:3, :].set(W2[:, :3].T)
    u, v = pl.pallas_call(
        _k2,
        grid=(N // BLK,),
        in_specs=[blk, blk, full(D, D), full(D, D), stspec,
                  rowspec, rowspec],
        out_specs=[blk, blk],
        out_shape=[jax.ShapeDtypeStruct((N, D), f32),
                   jax.ShapeDtypeStruct((N, D), f32)],
    )(h, p_pad, W2[:, 3:].T, w2xt, st1, row(g1), row(b1))

    qa = jnp.zeros((NP, 8), f32).at[:N, :3].set(p)
    pta = jnp.zeros((8, NP), f32).at[:3, :N].set(2.0 * p.T)
    pta = pta.at[3, N:].set(PAD_H)
    idx = pl.pallas_call(
        _k3,
        grid=(NP // BQ,),
        in_specs=[pl.BlockSpec((BQ, 8), lambda i: (i, 0)), full(8, NP)],
        out_specs=pl.BlockSpec((BQ, KNN), lambda i: (i, 0)),
        out_shape=jax.ShapeDtypeStruct((NP, KNN), jnp.int32),
    )(qa, pta)

    mx, mn, sm, sq = _gather_reduce(u, idx.reshape(-1))

    sums2 = pl.pallas_call(
        _k5,
        grid=(N // BLK,),
        in_specs=[blk, blk, blk],
        out_specs=stspec,
        out_shape=jax.ShapeDtypeStruct((8, D), f32),
    )(sm[:N], sq[:N], v)

    t, st3 = pl.pallas_call(
        _k6,
        grid=(N // BLK,),
        in_specs=[blk, blk, blk, full(D, D), stspec, rowspec, rowspec],
        out_specs=[blk, stspec],
        out_shape=[jax.ShapeDtypeStruct((N, D), f32),
                   jax.ShapeDtypeStruct((8, D), f32)],
    )(mx[:N], mn[:N], v, W3.T, sums2, row(g2), row(b2))

    out = pl.pallas_call(
        _k7,
        grid=(N // BLK,),
        in_specs=[blk, blk, stspec, rowspec, rowspec],
        out_specs=blk,
        out_shape=jax.ShapeDtypeStruct((N, D), f32),
    )(t, x, st3, row(g3), row(b3))

    return (p, out, o)
